# Initial kernel scaffold; baseline (speedup 1.0000x reference)
import jax
import jax.numpy as jnp
from jax.experimental import pallas as pl


def kernel(z, W):
    raise NotImplementedError("write your pallas kernel here")


# speedup vs baseline: 3.1450x; 3.1450x over previous
import functools

import jax
import jax.numpy as jnp
from jax.experimental import pallas as pl

_K = 512
_D = 128
_B = 16
_L = 4096
_LBLK = 512


def _vq_body(z_ref, w_ref, out_ref, loss_ref):
    b = pl.program_id(0)
    l = pl.program_id(1)

    zb = z_ref[0]
    w = w_ref[...]

    fz2 = jnp.sum(zb * zb, axis=0, keepdims=True)
    w2 = jnp.sum(w * w, axis=1, keepdims=True)
    dot = jax.lax.dot_general(
        w, zb, (((1,), (0,)), ((), ())),
        preferred_element_type=jnp.float32)
    d = (fz2 + w2) - 2.0 * dot

    minv = jnp.min(d, axis=0, keepdims=True)
    iota = jax.lax.broadcasted_iota(jnp.int32, d.shape, 0)
    idx = jnp.min(jnp.where(d == minv, iota, _K),
                  axis=0, keepdims=True)
    onehot = (iota == idx).astype(jnp.float32)

    q = jax.lax.dot_general(
        w, onehot, (((0,), (0,)), ((), ())),
        preferred_element_type=jnp.float32)
    out_ref[0] = q

    @pl.when((b == 0) & (l == 0))
    def _init():
        loss_ref[...] = jnp.zeros((1, 1), jnp.float32)

    loss_ref[...] += jnp.sum(minv, axis=1, keepdims=True)


@functools.partial(jax.jit, static_argnames=())
def kernel(z, W):
    grid = (_B, _L // _LBLK)
    out, loss = pl.pallas_call(
        _vq_body,
        grid=grid,
        in_specs=[
            pl.BlockSpec((1, _D, _LBLK), lambda b, l: (b, 0, l)),
            pl.BlockSpec((_K, _D), lambda b, l: (0, 0)),
        ],
        out_specs=[
            pl.BlockSpec((1, _D, _LBLK), lambda b, l: (b, 0, l)),
            pl.BlockSpec((1, 1), lambda b, l: (0, 0)),
        ],
        out_shape=[
            jax.ShapeDtypeStruct(z.shape, jnp.float32),
            jax.ShapeDtypeStruct((1, 1), jnp.float32),
        ],
    )(z, W)
    scale = 1.25 / (_B * _L * _D)
    return out, (loss[0, 0] * scale).astype(jnp.float32)


# speedup vs baseline: 4.6977x; 1.4937x over previous
import functools

import jax
import jax.numpy as jnp
from jax.experimental import pallas as pl

_K = 512
_D = 128
_B = 16
_L = 4096
_LBLK = 1024


def _vq_body(z_ref, w_ref, out_ref, loss_ref):
    b = pl.program_id(0)
    l = pl.program_id(1)

    zb = z_ref[0]
    w = w_ref[...]

    fz2 = jnp.sum(zb * zb, axis=0, keepdims=True)
    w2 = jnp.sum(w * w, axis=1, keepdims=True)
    dot = jax.lax.dot_general(
        w, zb, (((1,), (0,)), ((), ())),
        preferred_element_type=jnp.float32)
    d = (fz2 + w2) - 2.0 * dot

    minv = jnp.min(d, axis=0, keepdims=True)
    iota = jax.lax.broadcasted_iota(jnp.int32, d.shape, 0)
    idx = jnp.min(jnp.where(d == minv, iota, _K),
                  axis=0, keepdims=True)
    onehot = (iota == idx).astype(jnp.float32)

    q = jax.lax.dot_general(
        w, onehot, (((0,), (0,)), ((), ())),
        preferred_element_type=jnp.float32)
    out_ref[0] = q

    @pl.when((b == 0) & (l == 0))
    def _init():
        loss_ref[...] = jnp.zeros((1, 1), jnp.float32)

    loss_ref[...] += jnp.sum(minv, axis=1, keepdims=True)


@functools.partial(jax.jit, static_argnames=())
def kernel(z, W):
    grid = (_B, _L // _LBLK)
    out, loss = pl.pallas_call(
        _vq_body,
        grid=grid,
        in_specs=[
            pl.BlockSpec((1, _D, _LBLK), lambda b, l: (b, 0, l)),
            pl.BlockSpec((_K, _D), lambda b, l: (0, 0)),
        ],
        out_specs=[
            pl.BlockSpec((1, _D, _LBLK), lambda b, l: (b, 0, l)),
            pl.BlockSpec((1, 1), lambda b, l: (0, 0)),
        ],
        out_shape=[
            jax.ShapeDtypeStruct(z.shape, jnp.float32),
            jax.ShapeDtypeStruct((1, 1), jnp.float32),
        ],
    )(z, W)
    scale = 1.25 / (_B * _L * _D)
    return out, (loss[0, 0] * scale).astype(jnp.float32)


# speedup vs baseline: 5.6381x; 1.2002x over previous
import functools

import jax
import jax.numpy as jnp
from jax.experimental import pallas as pl

_K = 512
_D = 128
_B = 16
_L = 4096
_LBLK = 2048


def _vq_body(z_ref, w_ref, out_ref, loss_ref):
    b = pl.program_id(0)
    l = pl.program_id(1)

    zb = z_ref[0]
    w = w_ref[...]

    fz2 = jnp.sum(zb * zb, axis=0, keepdims=True)
    w2 = jnp.sum(w * w, axis=1, keepdims=True)
    dot = jax.lax.dot_general(
        w, zb, (((1,), (0,)), ((), ())),
        preferred_element_type=jnp.float32)
    d = (fz2 + w2) - 2.0 * dot

    minv = jnp.min(d, axis=0, keepdims=True)
    iota = jax.lax.broadcasted_iota(jnp.int32, d.shape, 0)
    idx = jnp.min(jnp.where(d == minv, iota, _K),
                  axis=0, keepdims=True)
    onehot = (iota == idx).astype(jnp.float32)

    q = jax.lax.dot_general(
        w, onehot, (((0,), (0,)), ((), ())),
        preferred_element_type=jnp.float32)
    out_ref[0] = q

    @pl.when((b == 0) & (l == 0))
    def _init():
        loss_ref[...] = jnp.zeros((1, 1), jnp.float32)

    loss_ref[...] += jnp.sum(minv, axis=1, keepdims=True)


@functools.partial(jax.jit, static_argnames=())
def kernel(z, W):
    grid = (_B, _L // _LBLK)
    out, loss = pl.pallas_call(
        _vq_body,
        grid=grid,
        in_specs=[
            pl.BlockSpec((1, _D, _LBLK), lambda b, l: (b, 0, l)),
            pl.BlockSpec((_K, _D), lambda b, l: (0, 0)),
        ],
        out_specs=[
            pl.BlockSpec((1, _D, _LBLK), lambda b, l: (b, 0, l)),
            pl.BlockSpec((1, 1), lambda b, l: (0, 0)),
        ],
        out_shape=[
            jax.ShapeDtypeStruct(z.shape, jnp.float32),
            jax.ShapeDtypeStruct((1, 1), jnp.float32),
        ],
    )(z, W)
    scale = 1.25 / (_B * _L * _D)
    return out, (loss[0, 0] * scale).astype(jnp.float32)


# speedup vs baseline: 7.2754x; 1.2904x over previous
import jax
import jax.numpy as jnp
from jax.experimental import pallas as pl

_K = 512
_D = 128
_B = 16
_L = 4096
_LBLK = 2048
_RC = 8


def _vq_body(z_ref, w_ref, out_ref, loss_ref):
    b = pl.program_id(0)
    l = pl.program_id(1)

    zb = z_ref[0]
    w = w_ref[...]

    fz2 = jnp.sum(zb * zb, axis=0, keepdims=True)
    w2 = jnp.sum(w * w, axis=1, keepdims=True)
    dot2 = jax.lax.dot_general(
        w * 2.0, zb, (((1,), (0,)), ((), ())),
        preferred_element_type=jnp.float32)

    accv = (fz2 + w2[0:_RC]) - dot2[0:_RC]
    acci = jnp.zeros((_RC, _LBLK), jnp.int32)
    for i in range(1, _K // _RC):
        dch = (fz2 + w2[i * _RC:(i + 1) * _RC]) - dot2[i * _RC:(i + 1) * _RC]
        lt = dch < accv
        accv = jnp.where(lt, dch, accv)
        acci = jnp.where(lt, i, acci)

    minv = jnp.min(accv, axis=0, keepdims=True)
    siota = jax.lax.broadcasted_iota(jnp.int32, (_RC, _LBLK), 0)
    fidx = acci * _RC + siota
    idx = jnp.min(jnp.where(accv == minv, fidx, _K),
                  axis=0, keepdims=True)
    kiota = jax.lax.broadcasted_iota(jnp.int32, (_K, _LBLK), 0)
    onehot = (kiota == idx).astype(jnp.float32)

    q = jax.lax.dot_general(
        w, onehot, (((0,), (0,)), ((), ())),
        preferred_element_type=jnp.float32)
    out_ref[0] = q

    @pl.when((b == 0) & (l == 0))
    def _init():
        loss_ref[...] = jnp.zeros((1, 1), jnp.float32)

    loss_ref[...] += jnp.sum(minv, axis=1, keepdims=True)


@jax.jit
def kernel(z, W):
    grid = (_B, _L // _LBLK)
    out, loss = pl.pallas_call(
        _vq_body,
        grid=grid,
        in_specs=[
            pl.BlockSpec((1, _D, _LBLK), lambda b, l: (b, 0, l)),
            pl.BlockSpec((_K, _D), lambda b, l: (0, 0)),
        ],
        out_specs=[
            pl.BlockSpec((1, _D, _LBLK), lambda b, l: (b, 0, l)),
            pl.BlockSpec((1, 1), lambda b, l: (0, 0)),
        ],
        out_shape=[
            jax.ShapeDtypeStruct(z.shape, jnp.float32),
            jax.ShapeDtypeStruct((1, 1), jnp.float32),
        ],
    )(z, W)
    scale = 1.25 / (_B * _L * _D)
    return out, (loss[0, 0] * scale).astype(jnp.float32)


# speedup vs baseline: 8.2568x; 1.1349x over previous
import jax
import jax.numpy as jnp
from jax.experimental import pallas as pl

_K = 512
_D = 128
_B = 16
_L = 4096
_LBLK = 4096
_RC = 8


def _vq_body(z_ref, w_ref, out_ref, loss_ref):
    b = pl.program_id(0)
    l = pl.program_id(1)

    zb = z_ref[0]
    w = w_ref[...]

    fz2 = jnp.sum(zb * zb, axis=0, keepdims=True)
    w2 = jnp.sum(w * w, axis=1, keepdims=True)
    dot2 = jax.lax.dot_general(
        w * 2.0, zb, (((1,), (0,)), ((), ())),
        preferred_element_type=jnp.float32)

    accv = (fz2 + w2[0:_RC]) - dot2[0:_RC]
    acci = jnp.zeros((_RC, _LBLK), jnp.int32)
    for i in range(1, _K // _RC):
        dch = (fz2 + w2[i * _RC:(i + 1) * _RC]) - dot2[i * _RC:(i + 1) * _RC]
        lt = dch < accv
        accv = jnp.where(lt, dch, accv)
        acci = jnp.where(lt, i, acci)

    minv = jnp.min(accv, axis=0, keepdims=True)
    siota = jax.lax.broadcasted_iota(jnp.int32, (_RC, _LBLK), 0)
    fidx = acci * _RC + siota
    idx = jnp.min(jnp.where(accv == minv, fidx, _K),
                  axis=0, keepdims=True)
    kiota = jax.lax.broadcasted_iota(jnp.int32, (_K, _LBLK), 0)
    onehot = (kiota == idx).astype(jnp.float32)

    q = jax.lax.dot_general(
        w, onehot, (((0,), (0,)), ((), ())),
        preferred_element_type=jnp.float32)
    out_ref[0] = q

    @pl.when((b == 0) & (l == 0))
    def _init():
        loss_ref[...] = jnp.zeros((1, 1), jnp.float32)

    loss_ref[...] += jnp.sum(minv, axis=1, keepdims=True)


@jax.jit
def kernel(z, W):
    grid = (_B, _L // _LBLK)
    out, loss = pl.pallas_call(
        _vq_body,
        grid=grid,
        in_specs=[
            pl.BlockSpec((1, _D, _LBLK), lambda b, l: (b, 0, l)),
            pl.BlockSpec((_K, _D), lambda b, l: (0, 0)),
        ],
        out_specs=[
            pl.BlockSpec((1, _D, _LBLK), lambda b, l: (b, 0, l)),
            pl.BlockSpec((1, 1), lambda b, l: (0, 0)),
        ],
        out_shape=[
            jax.ShapeDtypeStruct(z.shape, jnp.float32),
            jax.ShapeDtypeStruct((1, 1), jnp.float32),
        ],
    )(z, W)
    scale = 1.25 / (_B * _L * _D)
    return out, (loss[0, 0] * scale).astype(jnp.float32)
